# baseline (speedup 1.0000x reference)
import functools

import jax
import jax.numpy as jnp
from jax import lax
from jax.experimental import pallas as pl
from jax.experimental.pallas import tpu as pltpu

F32 = jnp.float32
BF16 = jnp.bfloat16
NEG_INF = float("-inf")


def _gelu(x):
    return 0.5 * x * (1.0 + lax.erf(x * (2.0 ** -0.5)))


def _moe_body(E, tokens_ref, task_ids_ref, task_embed_ref, gate_w_ref, gate_b_ref,
              w1_ref, b1_ref, w2_ref, b2_ref, uw1_ref, ub1_ref, uw2_ref, ub2_ref,
              out_ref, logits_ref, xbf_s, gates_s):
    e = pl.program_id(0)
    f = pl.program_id(1)
    N = tokens_ref.shape[1]
    D = tokens_ref.shape[2]

    @pl.when((e == 0) & (f == 0))
    def _gating():
        x = tokens_ref[0]
        xbf_s[...] = x.astype(BF16)
        tid = task_ids_ref[0]
        te = task_embed_ref[...]
        tlog = jnp.dot(te, gate_w_ref[D:, :])
        tio = lax.broadcasted_iota(jnp.int32, tlog.shape, 0)
        tsel = jnp.sum(jnp.where(tio == tid, tlog, 0.0), axis=0, keepdims=True)
        logits = (jnp.dot(x, gate_w_ref[:D, :])
                  + tsel + gate_b_ref[...][None, :])
        logits_ref[0] = logits
        io8 = lax.broadcasted_iota(jnp.int32, (N, E), 1)
        v1 = jnp.max(logits, axis=1, keepdims=True)
        i1 = jnp.min(jnp.where(logits == v1, io8, E), axis=1, keepdims=True)
        is1 = io8 == i1
        neg = jnp.where(is1, NEG_INF, logits)
        v2 = jnp.max(neg, axis=1, keepdims=True)
        i2 = jnp.min(jnp.where(neg == v2, io8, E), axis=1, keepdims=True)
        is2 = io8 == i2
        r = jnp.exp(v2 - v1)
        g1 = 1.0 / (1.0 + r)
        g2 = r / (1.0 + r)
        gate_t = jnp.where(is1, g1, jnp.where(is2, g2, 0.0))
        omega = 1.0 - g1
        pad = jnp.zeros((N, 16 - E - 1), F32)
        gates_s[...] = jnp.concatenate([gate_t, omega, pad], axis=1)

    io16 = lax.broadcasted_iota(jnp.int32, (N, 16), 1)
    gcol = jnp.sum(jnp.where(io16 == e, gates_s[...], 0.0), axis=1, keepdims=True)
    xb = xbf_s[...]
    first = (e == 0) & (f == 0)

    RC = 512
    NRC = N // RC

    def mlp_acc(w1b, b1b, w2b, b2b):
        w1c = w1b.astype(BF16)
        w2c = w2b.astype(BF16)
        for r in range(NRC):
            lo, hi = r * RC, (r + 1) * RC
            h = jnp.dot(xb[lo:hi, :], w1c, preferred_element_type=F32) + b1b[None, :]
            h = _gelu(h)
            y = jnp.dot(h.astype(BF16), w2c, preferred_element_type=F32)
            y = jnp.where(f == 0, y + b2b[None, :], y)
            val = gcol[lo:hi, :] * y

            @pl.when(first)
            def _init():
                out_ref[0, lo:hi, :] = val

            @pl.when(jnp.logical_not(first))
            def _add():
                out_ref[0, lo:hi, :] += val

    @pl.when(e < E)
    def _expert():
        mlp_acc(w1_ref[0], b1_ref[0, 0], w2_ref[0], b2_ref[0, 0])

    @pl.when(e == E)
    def _universal():
        mlp_acc(uw1_ref[...], ub1_ref[0, 0], uw2_ref[...], ub2_ref[0, 0])


def _moe_fused(tokens, task_ids, task_embed, gate_w, gate_b,
               w1, b1, w2, b2, uw1, ub1, uw2, ub2, *, interpret=False):
    B, N, D = tokens.shape
    E = gate_w.shape[1]
    T = task_embed.shape[0]
    F = w1.shape[2]
    FC = 768
    NF = F // FC
    grid = (E + 1, NF)
    elast = E - 1

    body = functools.partial(_moe_body, E)

    out, logits = pl.pallas_call(
        body,
        grid=grid,
        in_specs=[
            pl.BlockSpec((1, N, D), lambda e, f: (0, 0, 0)),
            pl.BlockSpec(memory_space=pltpu.SMEM),
            pl.BlockSpec((T, D), lambda e, f: (0, 0)),
            pl.BlockSpec((2 * D, E), lambda e, f: (0, 0)),
            pl.BlockSpec((E,), lambda e, f: (0,)),
            pl.BlockSpec((1, D, FC), lambda e, f: (jnp.minimum(e, elast), 0, f)),
            pl.BlockSpec((1, 1, FC), lambda e, f: (jnp.minimum(e, elast), 0, f)),
            pl.BlockSpec((1, FC, D), lambda e, f: (jnp.minimum(e, elast), f, 0)),
            pl.BlockSpec((1, 1, D), lambda e, f: (jnp.minimum(e, elast), 0, 0)),
            pl.BlockSpec((D, FC), lambda e, f: (0, f)),
            pl.BlockSpec((1, 1, FC), lambda e, f: (0, 0, f)),
            pl.BlockSpec((FC, D), lambda e, f: (f, 0)),
            pl.BlockSpec((1, 1, D), lambda e, f: (0, 0, 0)),
        ],
        out_specs=[
            pl.BlockSpec((1, N, D), lambda e, f: (0, 0, 0)),
            pl.BlockSpec((1, N, E), lambda e, f: (0, 0, 0)),
        ],
        out_shape=[
            jax.ShapeDtypeStruct((B, N, D), F32),
            jax.ShapeDtypeStruct((B, N, E), F32),
        ],
        scratch_shapes=[
            pltpu.VMEM((N, D), BF16),
            pltpu.VMEM((N, 16), F32),
        ],
        interpret=interpret,
    )(tokens, task_ids, task_embed, gate_w, gate_b,
      w1, b1.reshape(E, 1, F), w2, b2.reshape(E, 1, D),
      uw1, ub1.reshape(1, 1, F), uw2, ub2.reshape(1, 1, D))
    return out, logits


def kernel(tokens, task_ids, task_embed, gate_w, gate_b,
           w1, b1, w2, b2, uw1, ub1, uw2, ub2):
    return _moe_fused(tokens, task_ids, task_embed, gate_w, gate_b,
                      w1, b1, w2, b2, uw1, ub1, uw2, ub2)


# speedup vs baseline: 1.2137x; 1.2137x over previous
import functools

import jax
import jax.numpy as jnp
from jax import lax
from jax.experimental import pallas as pl
from jax.experimental.pallas import tpu as pltpu

F32 = jnp.float32
BF16 = jnp.bfloat16
NEG_INF = float("-inf")


def _gelu(x):
    return 0.5 * x * (1.0 + lax.erf(x * (2.0 ** -0.5)))


def _moe_body(E, tokens_ref, task_ids_ref, task_embed_ref, gate_w_ref, gate_b_ref,
              w1_ref, b1_ref, w2_ref, b2_ref, uw1_ref, ub1_ref, uw2_ref, ub2_ref,
              out_ref, logits_ref, xbf_s, gates_s):
    e = pl.program_id(0)
    f = pl.program_id(1)
    N = tokens_ref.shape[1]
    D = tokens_ref.shape[2]

    @pl.when((e == 0) & (f == 0))
    def _gating():
        x = tokens_ref[0]
        xbf_s[...] = x.astype(BF16)
        tid = task_ids_ref[0]
        te = task_embed_ref[...]
        tlog = jnp.dot(te, gate_w_ref[D:, :])
        tio = lax.broadcasted_iota(jnp.int32, tlog.shape, 0)
        tsel = jnp.sum(jnp.where(tio == tid, tlog, 0.0), axis=0, keepdims=True)
        logits = (jnp.dot(x, gate_w_ref[:D, :])
                  + tsel + gate_b_ref[...][None, :])
        logits_ref[0] = logits
        io8 = lax.broadcasted_iota(jnp.int32, (N, E), 1)
        v1 = jnp.max(logits, axis=1, keepdims=True)
        i1 = jnp.min(jnp.where(logits == v1, io8, E), axis=1, keepdims=True)
        is1 = io8 == i1
        neg = jnp.where(is1, NEG_INF, logits)
        v2 = jnp.max(neg, axis=1, keepdims=True)
        i2 = jnp.min(jnp.where(neg == v2, io8, E), axis=1, keepdims=True)
        is2 = io8 == i2
        r = jnp.exp(v2 - v1)
        g1 = 1.0 / (1.0 + r)
        g2 = r / (1.0 + r)
        gate_t = jnp.where(is1, g1, jnp.where(is2, g2, 0.0))
        omega = 1.0 - g1
        pad = jnp.zeros((N, 16 - E - 1), F32)
        gates_s[...] = jnp.concatenate([gate_t, omega, pad], axis=1)

    io16 = lax.broadcasted_iota(jnp.int32, (N, 16), 1)
    gcol = jnp.sum(jnp.where(io16 == e, gates_s[...], 0.0), axis=1, keepdims=True)
    xb = xbf_s[...]
    first = (e == 0) & (f == 0)

    def mlp_acc(w1b, b1b, w2b, b2b):
        h = jnp.dot(xb, w1b.astype(BF16), preferred_element_type=F32) + b1b[None, :]
        h = _gelu(h)
        y = jnp.dot(h.astype(BF16), w2b.astype(BF16), preferred_element_type=F32)
        y = jnp.where(f == 0, y + b2b[None, :], y)
        val = gcol * y

        @pl.when(first)
        def _init():
            out_ref[0] = val

        @pl.when(jnp.logical_not(first))
        def _add():
            out_ref[0] += val

    @pl.when(e < E)
    def _expert():
        mlp_acc(w1_ref[0], b1_ref[0, 0], w2_ref[0], b2_ref[0, 0])

    @pl.when(e == E)
    def _universal():
        mlp_acc(uw1_ref[...], ub1_ref[0, 0], uw2_ref[...], ub2_ref[0, 0])


def _moe_fused(tokens, task_ids, task_embed, gate_w, gate_b,
               w1, b1, w2, b2, uw1, ub1, uw2, ub2, *, interpret=False):
    B, N, D = tokens.shape
    E = gate_w.shape[1]
    T = task_embed.shape[0]
    F = w1.shape[2]
    FC = 1024
    NF = F // FC
    grid = (E + 1, NF)
    elast = E - 1

    body = functools.partial(_moe_body, E)

    out, logits = pl.pallas_call(
        body,
        grid=grid,
        in_specs=[
            pl.BlockSpec((1, N, D), lambda e, f: (0, 0, 0)),
            pl.BlockSpec(memory_space=pltpu.SMEM),
            pl.BlockSpec((T, D), lambda e, f: (0, 0)),
            pl.BlockSpec((2 * D, E), lambda e, f: (0, 0)),
            pl.BlockSpec((E,), lambda e, f: (0,)),
            pl.BlockSpec((1, D, FC), lambda e, f: (jnp.minimum(e, elast), 0, f)),
            pl.BlockSpec((1, 1, FC), lambda e, f: (jnp.minimum(e, elast), 0, f)),
            pl.BlockSpec((1, FC, D), lambda e, f: (jnp.minimum(e, elast), f, 0)),
            pl.BlockSpec((1, 1, D), lambda e, f: (jnp.minimum(e, elast), 0, 0)),
            pl.BlockSpec((D, FC), lambda e, f: (0, f)),
            pl.BlockSpec((1, 1, FC), lambda e, f: (0, 0, f)),
            pl.BlockSpec((FC, D), lambda e, f: (f, 0)),
            pl.BlockSpec((1, 1, D), lambda e, f: (0, 0, 0)),
        ],
        out_specs=[
            pl.BlockSpec((1, N, D), lambda e, f: (0, 0, 0)),
            pl.BlockSpec((1, N, E), lambda e, f: (0, 0, 0)),
        ],
        out_shape=[
            jax.ShapeDtypeStruct((B, N, D), F32),
            jax.ShapeDtypeStruct((B, N, E), F32),
        ],
        scratch_shapes=[
            pltpu.VMEM((N, D), BF16),
            pltpu.VMEM((N, 16), F32),
        ],
        interpret=interpret,
    )(tokens, task_ids, task_embed, gate_w, gate_b,
      w1, b1.reshape(E, 1, F), w2, b2.reshape(E, 1, D),
      uw1, ub1.reshape(1, 1, F), uw2, ub2.reshape(1, 1, D))
    return out, logits


def kernel(tokens, task_ids, task_embed, gate_w, gate_b,
           w1, b1, w2, b2, uw1, ub1, uw2, ub2):
    return _moe_fused(tokens, task_ids, task_embed, gate_w, gate_b,
                      w1, b1, w2, b2, uw1, ub1, uw2, ub2)
